# baseline (speedup 1.0000x reference)
import functools

import jax
import jax.numpy as jnp
from jax import lax
from jax.experimental import pallas as pl
from jax.experimental.pallas import tpu as pltpu
from jax.experimental.pallas import tpu_sc as plsc

N_FIELDS = 26
VOCAB = 100000
WIDTH = 32
BATCH = 16384

PACK = 4
ROWS4 = N_FIELDS * VOCAB // PACK
ROWS4_F = VOCAB // PACK

NC = 2
NS = 16
NW = NC * NS
BPW = BATCH // NW
BLK = 128
NBLK = BPW // BLK
G8 = 8
NG = BLK // G8
LANES = 16

_mesh = plsc.VectorSubcoreMesh(core_axis_name="c", subcore_axis_name="s")


@functools.partial(
    pl.kernel,
    out_type=jax.ShapeDtypeStruct((N_FIELDS, WIDTH, BATCH), jnp.float32),
    mesh=_mesh,
    scratch_types=[
        pltpu.VMEM((BPW,), jnp.int32),
        pltpu.VMEM((BPW,), jnp.int32),
        pltpu.VMEM((BPW,), jnp.int32),
        pltpu.VMEM((NG, G8, 128), jnp.float32),
        pltpu.VMEM((PACK, 8, 128), jnp.float32),
        pltpu.SemaphoreType.DMA,
        pltpu.SemaphoreType.DMA,
    ],
    compiler_params=pltpu.CompilerParams(needs_layout_passes=False),
)
def _embed_gather(xs_hbm, tab4_hbm, out_hbm, xsv, idx4, subv, stage, tiles,
                  sem_g, sem_o):
    wid = lax.axis_index("s") * NC + lax.axis_index("c")
    base_b = wid * BPW
    lane = lax.iota(jnp.int32, LANES)

    def per_field(f, carry):
        pltpu.sync_copy(xs_hbm.at[pl.ds(f * BATCH + base_b, BPW)], xsv)
        fbase = f * ROWS4_F

        def prep(i, c):
            sl = pl.ds(i * LANES, LANES)
            v = xsv[sl]
            idx4[sl] = fbase + (v >> 2)
            subv[sl] = (v & 3) * WIDTH
            return c

        lax.fori_loop(0, BPW // LANES, prep, 0)

        def per_block(blk, c):
            def fire(g, c2):
                pltpu.async_copy(
                    tab4_hbm.at[idx4.at[pl.ds(blk * BLK + g * G8, G8)]],
                    stage.at[g],
                    sem_g,
                )
                return c2

            lax.fori_loop(0, NG, fire, 0)

            def drain(g, c2):
                pltpu.make_async_copy(
                    tab4_hbm.at[idx4.at[pl.ds(blk * BLK + g * G8, G8)]],
                    stage.at[g],
                    sem_g,
                ).wait()
                return c2

            lax.fori_loop(0, NG, drain, 0)

            def asm(kk, c2):
                k16 = kk * LANES + lane
                i0 = k16 >> 3
                i1 = k16 & 7
                sub16 = subv[pl.ds(blk * BLK + kk * LANES, LANES)]
                for w in range(WIDTH):
                    val = plsc.load_gather(stage, [i0, i1, sub16 + w])
                    tiles[w // 8, w % 8, pl.ds(kk * LANES, LANES)] = val
                return c2

            lax.fori_loop(0, BLK // LANES, asm, 0)

            b0 = base_b + blk * BLK
            for w8 in range(PACK):
                pltpu.async_copy(
                    tiles.at[w8],
                    out_hbm.at[f, pl.ds(w8 * 8, 8), pl.ds(b0, BLK)],
                    sem_o,
                )
            for w8 in range(PACK):
                pltpu.make_async_copy(
                    tiles.at[w8],
                    out_hbm.at[f, pl.ds(w8 * 8, 8), pl.ds(b0, BLK)],
                    sem_o,
                ).wait()
            return c

        lax.fori_loop(0, NBLK, per_block, 0)
        return carry

    lax.fori_loop(0, N_FIELDS, per_field, 0)


def kernel(xs, tables):
    xs_flat = xs.reshape(N_FIELDS * BATCH)
    tab4 = tables.reshape(ROWS4, PACK * WIDTH)
    out_t = _embed_gather(xs_flat, tab4)
    return jnp.transpose(out_t, (0, 2, 1))
